# baseline (speedup 1.0000x reference)
import functools

import jax
import jax.numpy as jnp
from jax import lax
from jax.experimental import pallas as pl
from jax.experimental.pallas import tpu as pltpu
from jax.experimental.pallas import tpu_sc as plsc

_NC, _NS = 2, 16
_CHUNK = 128
_ROW = 8


def _sc_gather(tab, sidx2, oidx3, n_smpl):
    B = tab.shape[0]
    ns_chunks, nc = sidx2.shape
    no_chunks = oidx3.shape[1]
    A_S = ns_chunks * nc
    A_O = no_chunks * nc

    mesh = plsc.VectorSubcoreMesh(core_axis_name="c", subcore_axis_name="s")

    @functools.partial(
        pl.kernel,
        out_type=(
            jax.ShapeDtypeStruct((B, A_S, _ROW), jnp.float32),
            jax.ShapeDtypeStruct((B, A_O, _ROW), jnp.float32),
        ),
        mesh=mesh,
        scratch_types=[
            pltpu.VMEM((ns_chunks, nc), jnp.int32),
            pltpu.VMEM((no_chunks, nc), jnp.int32),
            pltpu.VMEM((A_S, _ROW), jnp.float32),
            pltpu.VMEM((A_O, _ROW), jnp.float32),
            pltpu.SemaphoreType.DMA,
        ],
        compiler_params=pltpu.CompilerParams(use_tc_tiling_on_sc=False),
    )
    def gather_kernel(tab_hbm, sidx_hbm, oidx_hbm,
                      out_s_hbm, out_o_hbm,
                      sidx_vm, oidx_vm, srows_vm, orows_vm, sem):
        b = lax.axis_index("s") * _NC + lax.axis_index("c")
        pltpu.sync_copy(sidx_hbm, sidx_vm)
        pltpu.sync_copy(oidx_hbm.at[b], oidx_vm)

        off = jnp.full((16,), n_smpl, jnp.int32)
        for j in range(no_chunks):
            for k in range(nc // 16):
                v = oidx_vm[j, pl.ds(k * 16, 16)]
                oidx_vm[j, pl.ds(k * 16, 16)] = lax.add(v, off)

        copies = []
        for j in range(ns_chunks):
            copies.append(pltpu.async_copy(
                tab_hbm.at[b].at[sidx_vm.at[j]],
                srows_vm.at[pl.ds(j * nc, nc)], sem))
        for j in range(no_chunks):
            copies.append(pltpu.async_copy(
                tab_hbm.at[b].at[oidx_vm.at[j]],
                orows_vm.at[pl.ds(j * nc, nc)], sem))
        for c in copies:
            c.wait()

        pltpu.sync_copy(srows_vm, out_s_hbm.at[b])
        pltpu.sync_copy(orows_vm, out_o_hbm.at[b])

    return gather_kernel(tab, sidx2, oidx3)


def _tc_loss_sum(rel3, smpl_rows, obj_g3, tile):
    B, A_S, L3 = rel3.shape
    nt = A_S // tile

    def body(rel_ref, smpl_ref, obj_ref, out_ref):
        step = pl.program_id(0) * nt + pl.program_id(1)
        rel = rel_ref[0]
        smpl = smpl_ref[0]
        obj = obj_ref[0]
        col = lax.broadcasted_iota(jnp.int32, (tile, 8), 1)
        smpl_aug = jnp.where(col == 3, 1.0, smpl)
        lane = lax.broadcasted_iota(jnp.int32, (8, L3), 1)
        row = lax.broadcasted_iota(jnp.int32, (8, L3), 0)
        lane_c = lane - (lane // 3) * 3
        m8 = jnp.where(row == 3, obj, jnp.where(row == lane_c, -1.0, 0.0))
        base = lax.dot_general(
            smpl_aug, m8, (((1,), (0,)), ((), ())),
            preferred_element_type=jnp.float32)
        absdiff = jnp.abs(base - rel)
        ones = jnp.ones((L3, 1), jnp.float32)
        psum = jnp.sum(jax.lax.dot_general(
            absdiff, ones, (((1,), (0,)), ((), ())),
            preferred_element_type=jnp.float32))

        @pl.when(step == 0)
        def _():
            out_ref[0, 0] = psum

        @pl.when(step != 0)
        def _():
            out_ref[0, 0] += psum

    return pl.pallas_call(
        body,
        grid=(B, nt),
        in_specs=[
            pl.BlockSpec((1, tile, L3), lambda b, i: (b, i, 0)),
            pl.BlockSpec((1, tile, _ROW), lambda b, i: (b, i, 0)),
            pl.BlockSpec((1, 1, L3), lambda b, i: (b, 0, 0)),
        ],
        out_specs=pl.BlockSpec(memory_space=pltpu.SMEM),
        out_shape=jax.ShapeDtypeStruct((1, 1), jnp.float32),
        compiler_params=pltpu.CompilerParams(
            dimension_semantics=("arbitrary", "arbitrary")),
    )(rel3, smpl_rows, obj_g3)


def kernel(smpl_v_orig, object_v_orig, rel_dist, smpl_anchor_indices,
           object_anchor_indices):
    B, n_smpl, _ = smpl_v_orig.shape
    n_obj = object_v_orig.shape[1]
    A_S = smpl_anchor_indices.shape[0]
    A_O = object_anchor_indices.shape[1]

    tab = jnp.pad(
        jnp.concatenate([smpl_v_orig, object_v_orig], axis=1),
        ((0, 0), (0, 0), (0, _ROW - 3)))

    sidx2 = smpl_anchor_indices.astype(jnp.int32).reshape(
        A_S // _CHUNK, _CHUNK)
    oidx3 = object_anchor_indices.astype(jnp.int32).reshape(
        B, A_O // _CHUNK, _CHUNK)

    smpl_rows, obj_rows = _sc_gather(tab, sidx2, oidx3, n_smpl)

    rel3 = rel_dist.reshape(B, A_S, A_O * 3)
    obj_g3 = obj_rows[:, :, :3].reshape(B, 1, A_O * 3)

    total = _tc_loss_sum(rel3, smpl_rows, obj_g3, tile=1024)
    count = B * A_S * A_O * 3
    return (total / count).reshape(())


# speedup vs baseline: 5.3408x; 5.3408x over previous
import functools

import jax
import jax.numpy as jnp
from jax import lax
from jax.experimental import pallas as pl
from jax.experimental.pallas import tpu as pltpu
from jax.experimental.pallas import tpu_sc as plsc

_NC, _NS = 2, 16
_CHUNK = 128
_G = 8


def _granule_rows(n_vertices):
    return (3 * (n_vertices - 1)) // _G + 2


def _sc_gather_pairs(sflat3, oflat3, sidx2, oidx3):
    B = sflat3.shape[0]
    ns_chunks, nc = sidx2.shape
    no_chunks = oidx3.shape[1]
    A_S = ns_chunks * nc
    A_O = no_chunks * nc

    mesh = plsc.VectorSubcoreMesh(core_axis_name="c", subcore_axis_name="s")

    @functools.partial(
        pl.kernel,
        out_type=(
            jax.ShapeDtypeStruct((B, 2, A_S, _G), jnp.float32),
            jax.ShapeDtypeStruct((B, 2, A_O, _G), jnp.float32),
        ),
        mesh=mesh,
        scratch_types=[
            pltpu.VMEM((ns_chunks, nc), jnp.int32),
            pltpu.VMEM((no_chunks, nc), jnp.int32),
            pltpu.VMEM((ns_chunks, nc), jnp.int32),
            pltpu.VMEM((ns_chunks, nc), jnp.int32),
            pltpu.VMEM((no_chunks, nc), jnp.int32),
            pltpu.VMEM((no_chunks, nc), jnp.int32),
            pltpu.VMEM((A_S, _G), jnp.float32),
            pltpu.VMEM((A_S, _G), jnp.float32),
            pltpu.VMEM((A_O, _G), jnp.float32),
            pltpu.VMEM((A_O, _G), jnp.float32),
            pltpu.SemaphoreType.DMA,
        ],
        compiler_params=pltpu.CompilerParams(use_tc_tiling_on_sc=False),
    )
    def gather_kernel(sflat_hbm, oflat_hbm, sidx_hbm, oidx_hbm,
                      out_s_hbm, out_o_hbm,
                      sidx_vm, oidx_vm, gsa_vm, gsb_vm, goa_vm, gob_vm,
                      sra_vm, srb_vm, ora_vm, orb_vm, sem):
        b = lax.axis_index("s") * _NC + lax.axis_index("c")
        pltpu.sync_copy(sidx_hbm, sidx_vm)
        pltpu.sync_copy(oidx_hbm.at[b], oidx_vm)

        three = jnp.full((16,), 3, jnp.int32)
        shift3 = jnp.full((16,), 3, jnp.int32)
        one = jnp.full((16,), 1, jnp.int32)

        def make_rows(idx_vm, ga_vm, gb_vm, nchunks):
            for j in range(nchunks):
                for k in range(nc // 16):
                    sl = pl.ds(k * 16, 16)
                    v = idx_vm[j, sl]
                    g = lax.shift_right_logical(lax.mul(v, three), shift3)
                    ga_vm[j, sl] = g
                    gb_vm[j, sl] = lax.add(g, one)

        make_rows(sidx_vm, gsa_vm, gsb_vm, ns_chunks)
        make_rows(oidx_vm, goa_vm, gob_vm, no_chunks)

        copies = []
        for gv, dst, nch, tab in (
                (gsa_vm, sra_vm, ns_chunks, sflat_hbm),
                (gsb_vm, srb_vm, ns_chunks, sflat_hbm),
                (goa_vm, ora_vm, no_chunks, oflat_hbm),
                (gob_vm, orb_vm, no_chunks, oflat_hbm)):
            for j in range(nch):
                copies.append(pltpu.async_copy(
                    tab.at[b].at[gv.at[j]],
                    dst.at[pl.ds(j * nc, nc)], sem))
        for c in copies:
            c.wait()

        pltpu.sync_copy(sra_vm, out_s_hbm.at[b, 0])
        pltpu.sync_copy(srb_vm, out_s_hbm.at[b, 1])
        pltpu.sync_copy(ora_vm, out_o_hbm.at[b, 0])
        pltpu.sync_copy(orb_vm, out_o_hbm.at[b, 1])

    return gather_kernel(sflat3, oflat3, sidx2, oidx3)


def _extract_xyz(pairs16, idx_i32, n):
    off = lax.bitwise_and(3 * idx_i32, jnp.full((n, 1), 7, jnp.int32))
    exts = []
    for c in range(3):
        acc = jnp.zeros((n, 1), jnp.float32)
        for q in range(8):
            acc = acc + jnp.where(off == q, pairs16[:, q + c:q + c + 1], 0.0)
        exts.append(acc)
    return exts


def _tc_loss_sum(rel3, spairs, opairs, sidxf, oidxf, tile):
    B, A_S, L3 = rel3.shape
    A_O = opairs.shape[2]
    nt = A_S // tile

    def body(rel_ref, sp_ref, op_ref, si_ref, oi_ref, out_ref):
        step = pl.program_id(0) * nt + pl.program_id(1)
        rel = rel_ref[0]

        sidx = si_ref[0].astype(jnp.int32)
        spair16 = jnp.concatenate([sp_ref[0, 0], sp_ref[0, 1]], axis=1)
        se0, se1, se2 = _extract_xyz(spair16, sidx, tile)
        colv = lax.broadcasted_iota(jnp.int32, (tile, 8), 1)
        aug = jnp.where(colv == 0, se0,
                        jnp.where(colv == 1, se1,
                                  jnp.where(colv == 2, se2,
                                            jnp.where(colv == 3, 1.0, 0.0))))

        oidx = oi_ref[0].astype(jnp.int32)
        opair16 = jnp.concatenate([op_ref[0, 0], op_ref[0, 1]], axis=1)
        oe = _extract_xyz(opair16, oidx, A_O)
        orow = lax.broadcasted_iota(jnp.int32, (A_O, L3), 0)
        olane = lax.broadcasted_iota(jnp.int32, (A_O, L3), 1)
        qrow = (orow == olane // 3).astype(jnp.float32)
        lane1 = lax.broadcasted_iota(jnp.int32, (1, L3), 1)
        lane_c = lane1 - (lane1 // 3) * 3
        obj_int = jnp.zeros((1, L3), jnp.float32)
        for c in range(3):
            rep = lax.dot_general(oe[c], qrow, (((0,), (0,)), ((), ())),
                                  preferred_element_type=jnp.float32)
            obj_int = obj_int + rep * (lane_c == c).astype(jnp.float32)

        rowv = lax.broadcasted_iota(jnp.int32, (8, L3), 0)
        lanec8 = lax.broadcasted_iota(jnp.int32, (8, L3), 1)
        lanec8 = lanec8 - (lanec8 // 3) * 3
        m8 = jnp.where(rowv == 3, obj_int,
                       jnp.where(rowv == lanec8, -1.0, 0.0))
        base = lax.dot_general(aug, m8, (((1,), (0,)), ((), ())),
                               preferred_element_type=jnp.float32)
        absdiff = jnp.abs(base - rel)
        ones = jnp.ones((L3, 1), jnp.float32)
        psum = jnp.sum(lax.dot_general(
            absdiff, ones, (((1,), (0,)), ((), ())),
            preferred_element_type=jnp.float32))

        @pl.when(step == 0)
        def _():
            out_ref[0, 0] = psum

        @pl.when(step != 0)
        def _():
            out_ref[0, 0] += psum

    return pl.pallas_call(
        body,
        grid=(B, nt),
        in_specs=[
            pl.BlockSpec((1, tile, L3), lambda b, i: (b, i, 0)),
            pl.BlockSpec((1, 2, tile, _G), lambda b, i: (b, 0, i, 0)),
            pl.BlockSpec((1, 2, A_O, _G), lambda b, i: (b, 0, 0, 0)),
            pl.BlockSpec((1, tile, 1), lambda b, i: (0, i, 0)),
            pl.BlockSpec((1, A_O, 1), lambda b, i: (b, 0, 0)),
        ],
        out_specs=pl.BlockSpec(memory_space=pltpu.SMEM),
        out_shape=jax.ShapeDtypeStruct((1, 1), jnp.float32),
        compiler_params=pltpu.CompilerParams(
            dimension_semantics=("arbitrary", "arbitrary")),
    )(rel3, spairs, opairs, sidxf, oidxf)


def kernel(smpl_v_orig, object_v_orig, rel_dist, smpl_anchor_indices,
           object_anchor_indices):
    B, n_smpl, _ = smpl_v_orig.shape
    n_obj = object_v_orig.shape[1]
    A_S = smpl_anchor_indices.shape[0]
    A_O = object_anchor_indices.shape[1]

    ns8 = _granule_rows(n_smpl)
    no8 = _granule_rows(n_obj)
    sflat = jnp.pad(smpl_v_orig.reshape(B, -1),
                    ((0, 0), (0, ns8 * _G - 3 * n_smpl)))
    oflat = jnp.pad(object_v_orig.reshape(B, -1),
                    ((0, 0), (0, no8 * _G - 3 * n_obj)))
    sflat3 = sflat.reshape(B, ns8, _G)
    oflat3 = oflat.reshape(B, no8, _G)

    sidx = smpl_anchor_indices.astype(jnp.int32)
    oidx = object_anchor_indices.astype(jnp.int32)
    sidx2 = sidx.reshape(A_S // _CHUNK, _CHUNK)
    oidx3 = oidx.reshape(B, A_O // _CHUNK, _CHUNK)

    spairs, opairs = _sc_gather_pairs(sflat3, oflat3, sidx2, oidx3)

    rel3 = rel_dist.reshape(B, A_S, A_O * 3)
    sidxf = sidx.astype(jnp.float32).reshape(1, A_S, 1)
    oidxf = oidx.astype(jnp.float32).reshape(B, A_O, 1)

    total = _tc_loss_sum(rel3, spairs, opairs, sidxf, oidxf, tile=1024)
    count = B * A_S * A_O * 3
    return (total / count).reshape(())
